# baseline (device time: 48248 ns/iter reference)
import jax
import jax.numpy as jnp
from jax import lax
from jax.experimental import pallas as pl
from jax.experimental.pallas import tpu as pltpu

N_DEV = 32
M = 768
N = 768
K1 = 1536
CH = M // N_DEV
NB = 4
BR = M // NB
CPB = N_DEV // NB


def kernel(x, W1, W2):
    def body(x_hbm, w1_hbm, w2_hbm, out_ref,
             xv, w1v, w2v, send_buf, recv1, recv2,
             in_sems, send_sems1, send_sems2, recv1_sems, recv2_sems):
        me = lax.axis_index("i")
        my_grp = me // CPB
        my_q = me % CPB

        cp_x = pltpu.make_async_copy(x_hbm, xv, in_sems.at[0])
        cp_w1 = pltpu.make_async_copy(w1_hbm, w1v, in_sems.at[1])
        cp_w2 = pltpu.make_async_copy(w2_hbm, w2v, in_sems.at[2])
        cp_x.start()
        cp_w1.start()
        cp_w2.start()

        bsem = pltpu.get_barrier_semaphore()
        for j in range(CPB):
            c = my_grp * CPB + j
            @pl.when(c != me)
            def _(c=c):
                pl.semaphore_signal(
                    bsem, inc=1,
                    device_id=(c,), device_id_type=pl.DeviceIdType.MESH,
                )

        cp_w1.wait()
        cp_x.wait()
        w1 = w1v[...].astype(jnp.bfloat16)

        bq0 = my_grp
        x0 = xv[pl.ds(bq0 * BR, BR), :].astype(jnp.bfloat16)
        h0 = jnp.dot(x0, w1, preferred_element_type=jnp.float32)
        h0b = jnp.maximum(h0, 0.0).astype(jnp.bfloat16)

        pl.semaphore_wait(bsem, CPB - 1)
        for g in range(1, NB):
            c = ((my_grp + g) % NB) * CPB + my_q
            pl.semaphore_signal(
                bsem, inc=1,
                device_id=(c,), device_id_type=pl.DeviceIdType.MESH,
            )

        cp_w2.wait()
        w2 = w2v[...].astype(jnp.bfloat16)
        p0 = jnp.dot(h0b, w2, preferred_element_type=jnp.float32)
        p0b = p0.astype(jnp.bfloat16)

        pl.semaphore_wait(bsem, NB - 1)

        def scatter_block(bq, pkb):
            for j in range(CPB):
                c = bq * CPB + j
                send_buf[c] = pkb[j * CH:(j + 1) * CH, :]
                @pl.when(c != me)
                def _(c=c):
                    pltpu.make_async_remote_copy(
                        src_ref=send_buf.at[c],
                        dst_ref=recv1.at[me],
                        send_sem=send_sems1.at[c],
                        recv_sem=recv1_sems.at[me],
                        device_id=(c,),
                        device_id_type=pl.DeviceIdType.MESH,
                    ).start()

        scatter_block(bq0, p0b)

        for k in range(1, NB):
            bq = (my_grp + k) % NB
            xk = xv[pl.ds(bq * BR, BR), :].astype(jnp.bfloat16)
            hk = jnp.dot(xk, w1, preferred_element_type=jnp.float32)
            hkb = jnp.maximum(hk, 0.0).astype(jnp.bfloat16)
            pk = jnp.dot(hkb, w2, preferred_element_type=jnp.float32)
            scatter_block(bq, pk.astype(jnp.bfloat16))

        recv1[me] = send_buf[me]
        red = recv1[me].astype(jnp.float32)
        for k in range(NB):
            for j in range(CPB):
                s = ((my_grp - k) % NB) * CPB + j
                @pl.when(s != me)
                def _(s=s):
                    pltpu.make_async_remote_copy(
                        src_ref=recv1.at[s],
                        dst_ref=recv1.at[s],
                        send_sem=send_sems1.at[s],
                        recv_sem=recv1_sems.at[s],
                        device_id=(s,),
                        device_id_type=pl.DeviceIdType.MESH,
                    ).wait_recv()
                zero = jnp.zeros((CH, N), jnp.float32)
                contrib = jnp.where(s == me, zero, recv1[s].astype(jnp.float32))
                red = red + contrib

        recv2[me] = red.astype(jnp.bfloat16)
        for c in range(N_DEV):
            @pl.when(me != c)
            def _(c=c):
                pltpu.make_async_remote_copy(
                    src_ref=recv2.at[me],
                    dst_ref=recv2.at[me],
                    send_sem=send_sems2.at[c],
                    recv_sem=recv2_sems.at[me],
                    device_id=(c,),
                    device_id_type=pl.DeviceIdType.MESH,
                ).start()

        out_ref[pl.ds(me * CH, CH), :] = red
        for s in range(N_DEV):
            @pl.when(me != s)
            def _(s=s):
                pltpu.make_async_remote_copy(
                    src_ref=recv2.at[s],
                    dst_ref=recv2.at[s],
                    send_sem=send_sems2.at[s],
                    recv_sem=recv2_sems.at[s],
                    device_id=(s,),
                    device_id_type=pl.DeviceIdType.MESH,
                ).wait_recv()
                out_ref[s * CH:(s + 1) * CH, :] = recv2[s].astype(jnp.float32)

        for c in range(N_DEV):
            @pl.when(me != c)
            def _(c=c):
                pltpu.make_async_remote_copy(
                    src_ref=send_buf.at[c],
                    dst_ref=recv1.at[me],
                    send_sem=send_sems1.at[c],
                    recv_sem=recv1_sems.at[me],
                    device_id=(c,),
                    device_id_type=pl.DeviceIdType.MESH,
                ).wait_send()
                pltpu.make_async_remote_copy(
                    src_ref=recv2.at[me],
                    dst_ref=recv2.at[me],
                    send_sem=send_sems2.at[c],
                    recv_sem=recv2_sems.at[me],
                    device_id=(c,),
                    device_id_type=pl.DeviceIdType.MESH,
                ).wait_send()

    return pl.pallas_call(
        body,
        out_shape=jax.ShapeDtypeStruct((M, N), jnp.float32),
        in_specs=[
            pl.BlockSpec(memory_space=pl.ANY),
            pl.BlockSpec(memory_space=pl.ANY),
            pl.BlockSpec(memory_space=pl.ANY),
        ],
        out_specs=pl.BlockSpec(memory_space=pltpu.VMEM),
        scratch_shapes=[
            pltpu.VMEM((M, N), jnp.float32),
            pltpu.VMEM((M, K1), jnp.float32),
            pltpu.VMEM((K1, N), jnp.float32),
            pltpu.VMEM((N_DEV, CH, N), jnp.bfloat16),
            pltpu.VMEM((N_DEV, CH, N), jnp.bfloat16),
            pltpu.VMEM((N_DEV, CH, N), jnp.bfloat16),
            pltpu.SemaphoreType.DMA((3,)),
            pltpu.SemaphoreType.DMA((N_DEV,)),
            pltpu.SemaphoreType.DMA((N_DEV,)),
            pltpu.SemaphoreType.DMA((N_DEV,)),
            pltpu.SemaphoreType.DMA((N_DEV,)),
        ],
        compiler_params=pltpu.CompilerParams(collective_id=0),
    )(x, W1, W2)


# device time: 45965 ns/iter; 1.0497x vs baseline; 1.0497x over previous
import jax
import jax.numpy as jnp
from jax import lax
from jax.experimental import pallas as pl
from jax.experimental.pallas import tpu as pltpu

N_DEV = 32
M = 768
N = 768
CH = M // N_DEV
NB = 4
BR = M // NB
CPB = N_DEV // NB


def kernel(x, W1, W2):
    def body(x_ref, w1_ref, w2_ref, out_ref,
             send_buf, recv1, recv2,
             send_sems1, send_sems2, recv1_sems, recv2_sems):
        me = lax.axis_index("i")
        my_grp = me // CPB

        bsem = pltpu.get_barrier_semaphore()
        for c in range(N_DEV):
            @pl.when(me != c)
            def _(c=c):
                pl.semaphore_signal(
                    bsem, inc=1,
                    device_id=(c,), device_id_type=pl.DeviceIdType.MESH,
                )

        w1 = w1_ref[...].astype(jnp.bfloat16)
        w2 = w2_ref[...].astype(jnp.bfloat16)

        for k in range(NB):
            bq = (my_grp + k) % NB
            r0 = bq * BR
            xk = x_ref[pl.ds(r0, BR), :].astype(jnp.bfloat16)
            hk = jnp.dot(xk, w1, preferred_element_type=jnp.float32)
            hkb = jnp.maximum(hk, 0.0).astype(jnp.bfloat16)
            pk = jnp.dot(hkb, w2, preferred_element_type=jnp.float32)
            pkb = pk.astype(jnp.bfloat16)
            if k == 0:
                pl.semaphore_wait(bsem, N_DEV - 1)
            for j in range(CPB):
                c = bq * CPB + j
                send_buf[c] = pkb[j * CH:(j + 1) * CH, :]
                @pl.when(c != me)
                def _(c=c):
                    pltpu.make_async_remote_copy(
                        src_ref=send_buf.at[c],
                        dst_ref=recv1.at[me],
                        send_sem=send_sems1.at[c],
                        recv_sem=recv1_sems.at[me],
                        device_id=(c,),
                        device_id_type=pl.DeviceIdType.MESH,
                    ).start()

        recv1[me] = send_buf[me]
        red = recv1[me].astype(jnp.float32)
        for k in range(NB):
            for j in range(CPB):
                s = ((my_grp - k) % NB) * CPB + j
                @pl.when(s != me)
                def _(s=s):
                    pltpu.make_async_remote_copy(
                        src_ref=recv1.at[s],
                        dst_ref=recv1.at[s],
                        send_sem=send_sems1.at[s],
                        recv_sem=recv1_sems.at[s],
                        device_id=(s,),
                        device_id_type=pl.DeviceIdType.MESH,
                    ).wait_recv()
                zero = jnp.zeros((CH, N), jnp.float32)
                contrib = jnp.where(s == me, zero, recv1[s].astype(jnp.float32))
                red = red + contrib

        recv2[me] = red.astype(jnp.bfloat16)
        for c in range(N_DEV):
            @pl.when(me != c)
            def _(c=c):
                pltpu.make_async_remote_copy(
                    src_ref=recv2.at[me],
                    dst_ref=recv2.at[me],
                    send_sem=send_sems2.at[c],
                    recv_sem=recv2_sems.at[me],
                    device_id=(c,),
                    device_id_type=pl.DeviceIdType.MESH,
                ).start()

        out_ref[pl.ds(me * CH, CH), :] = red
        for s in range(N_DEV):
            @pl.when(me != s)
            def _(s=s):
                pltpu.make_async_remote_copy(
                    src_ref=recv2.at[s],
                    dst_ref=recv2.at[s],
                    send_sem=send_sems2.at[s],
                    recv_sem=recv2_sems.at[s],
                    device_id=(s,),
                    device_id_type=pl.DeviceIdType.MESH,
                ).wait_recv()
                out_ref[s * CH:(s + 1) * CH, :] = recv2[s].astype(jnp.float32)

        for c in range(N_DEV):
            @pl.when(me != c)
            def _(c=c):
                pltpu.make_async_remote_copy(
                    src_ref=send_buf.at[c],
                    dst_ref=recv1.at[me],
                    send_sem=send_sems1.at[c],
                    recv_sem=recv1_sems.at[me],
                    device_id=(c,),
                    device_id_type=pl.DeviceIdType.MESH,
                ).wait_send()
                pltpu.make_async_remote_copy(
                    src_ref=recv2.at[me],
                    dst_ref=recv2.at[me],
                    send_sem=send_sems2.at[c],
                    recv_sem=recv2_sems.at[me],
                    device_id=(c,),
                    device_id_type=pl.DeviceIdType.MESH,
                ).wait_send()

    return pl.pallas_call(
        body,
        out_shape=jax.ShapeDtypeStruct((M, N), jnp.float32),
        in_specs=[
            pl.BlockSpec(memory_space=pltpu.VMEM),
            pl.BlockSpec(memory_space=pltpu.VMEM),
            pl.BlockSpec(memory_space=pltpu.VMEM),
        ],
        out_specs=pl.BlockSpec(memory_space=pltpu.VMEM),
        scratch_shapes=[
            pltpu.VMEM((N_DEV, CH, N), jnp.bfloat16),
            pltpu.VMEM((N_DEV, CH, N), jnp.bfloat16),
            pltpu.VMEM((N_DEV, CH, N), jnp.bfloat16),
            pltpu.SemaphoreType.DMA((N_DEV,)),
            pltpu.SemaphoreType.DMA((N_DEV,)),
            pltpu.SemaphoreType.DMA((N_DEV,)),
            pltpu.SemaphoreType.DMA((N_DEV,)),
        ],
        compiler_params=pltpu.CompilerParams(collective_id=0),
    )(x, W1, W2)
